# baseline (speedup 1.0000x reference)
import functools

import jax
import jax.numpy as jnp
from jax import lax
from jax.experimental import pallas as pl
from jax.experimental.pallas import tpu as pltpu
from jax.experimental.pallas import tpu_sc as plsc

VOCAB = 1000000
EMBED_DIM = 64
BATCH = 4096
HIST = 200

NC, NS = 2, 16
NW = NC * NS
LANES = 16
CHUNK = 128
BLOCKS_PER_T = BATCH // CHUNK
N_CHUNKS = HIST * BLOCKS_PER_T
CH_PER_W = N_CHUNKS // NW
NBUF = 2
N_GROUPS = CH_PER_W // NBUF


def _gather_body(ids_t_hbm, table_hbm, out_hbm, idx0, idx1, rows0, rows1,
                 tr0, tr1, isem0, isem1, gsem0, gsem1, wsem0, wsem1):
    wid = lax.axis_index("s") * NC + lax.axis_index("c")
    c_base = wid * CH_PER_W
    idxs = (idx0, idx1)
    bufs = (rows0, rows1)
    trs = (tr0, tr1)
    isems = (isem0, isem1)
    gsems = (gsem0, gsem1)
    wsems = (wsem0, wsem1)

    def fire_idx(c, b):
        t = c // BLOCKS_PER_T
        b0 = (c % BLOCKS_PER_T) * CHUNK
        pltpu.async_copy(ids_t_hbm.at[t, pl.ds(b0, CHUNK)], idxs[b], isems[b])

    def fire_gather(b):
        pltpu.async_copy(table_hbm.at[idxs[b]], bufs[b], gsems[b])

    def drain_gather(b):
        pltpu.make_async_copy(table_hbm.at[pl.ds(0, CHUNK)], bufs[b],
                              gsems[b]).wait()

    def transpose(b):
        buf, tr = bufs[b], trs[b]
        col = lax.iota(jnp.int32, LANES)

        def e_step(e, carry):
            sb = e // 8
            s = e % 8
            e_vec = jnp.full((LANES,), e, jnp.int32)
            for cb in range(CHUNK // LANES):
                rows_idx = col + (cb * LANES)
                vals = plsc.load_gather(buf, [rows_idx, e_vec])
                tr[sb, s, pl.ds(cb * LANES, LANES)] = vals
            return carry

        lax.fori_loop(0, EMBED_DIM, e_step, 0)

    def fire_wb(c, b):
        t = c // BLOCKS_PER_T
        vb = c % BLOCKS_PER_T
        pltpu.async_copy(trs[b], out_hbm.at[t, :, vb], wsems[b])

    def drain_wb(b):
        pltpu.make_async_copy(trs[b], out_hbm.at[0, :, 0], wsems[b]).wait()

    for b in range(NBUF):
        fire_idx(c_base + b, b)
    for b in range(NBUF):
        pltpu.make_async_copy(ids_t_hbm.at[0, pl.ds(0, CHUNK)], idxs[b],
                              isems[b]).wait()
        fire_gather(b)

    def group_step(k, carry):
        for b in range(NBUF):
            c = c_base + k * NBUF + b
            drain_gather(b)

            @pl.when(k < N_GROUPS - 1)
            def _():
                fire_idx(c + NBUF, b)

            @pl.when(k > 0)
            def _():
                drain_wb(b)

            transpose(b)
            fire_wb(c, b)

            @pl.when(k < N_GROUPS - 1)
            def _():
                pltpu.make_async_copy(ids_t_hbm.at[0, pl.ds(0, CHUNK)],
                                      idxs[b], isems[b]).wait()
                fire_gather(b)
        return carry

    lax.fori_loop(0, N_GROUPS, group_step, 0)
    for b in range(NBUF):
        drain_wb(b)


def kernel(input_ids, table):
    ids_t = input_ids.T.astype(jnp.int32)
    mesh = plsc.VectorSubcoreMesh(core_axis_name="c", subcore_axis_name="s")
    run = functools.partial(
        pl.kernel,
        mesh=mesh,
        out_type=jax.ShapeDtypeStruct(
            (HIST, EMBED_DIM // 8, BLOCKS_PER_T, 8, CHUNK), jnp.float32),
        scratch_types=[
            pltpu.VMEM((CHUNK,), jnp.int32),
            pltpu.VMEM((CHUNK,), jnp.int32),
            pltpu.VMEM((CHUNK, EMBED_DIM), jnp.float32),
            pltpu.VMEM((CHUNK, EMBED_DIM), jnp.float32),
            pltpu.VMEM((EMBED_DIM // 8, 8, CHUNK), jnp.float32),
            pltpu.VMEM((EMBED_DIM // 8, 8, CHUNK), jnp.float32),
            pltpu.SemaphoreType.DMA,
            pltpu.SemaphoreType.DMA,
            pltpu.SemaphoreType.DMA,
            pltpu.SemaphoreType.DMA,
            pltpu.SemaphoreType.DMA,
            pltpu.SemaphoreType.DMA,
        ],
        compiler_params=pltpu.CompilerParams(use_tc_tiling_on_sc=False, needs_layout_passes=False),
    )(_gather_body)
    out5 = run(ids_t, table)
    return out5.transpose(2, 4, 0, 1, 3).reshape(BATCH, HIST, EMBED_DIM)


# speedup vs baseline: 1.1440x; 1.1440x over previous
import functools

import jax
import jax.numpy as jnp
from jax import lax
from jax.experimental import pallas as pl
from jax.experimental.pallas import tpu as pltpu
from jax.experimental.pallas import tpu_sc as plsc

VOCAB = 1000000
EMBED_DIM = 64
BATCH = 4096
HIST = 200

NC, NS = 2, 16
NW = NC * NS
LANES = 16
CHUNK = 128
BLOCKS_PER_T = BATCH // CHUNK
N_CHUNKS = HIST * BLOCKS_PER_T
CH_PER_W = N_CHUNKS // NW
NBUF = 2
N_GROUPS = CH_PER_W // NBUF
C_UNROLL = 4


def _gather_body(ids_t_hbm, table_hbm, out_hbm, idx0, idx1, rows0, rows1,
                 tr0, tr1, isem0, isem1, gsem0, gsem1, wsem0, wsem1):
    wid = lax.axis_index("s") * NC + lax.axis_index("c")
    c_base = wid * CH_PER_W
    idxs = (idx0, idx1)
    bufs = (rows0, rows1)
    trs = (tr0, tr1)
    isems = (isem0, isem1)
    gsems = (gsem0, gsem1)
    wsems = (wsem0, wsem1)

    def fire_idx(c, b):
        t = c // BLOCKS_PER_T
        b0 = (c % BLOCKS_PER_T) * CHUNK
        pltpu.async_copy(ids_t_hbm.at[t, pl.ds(b0, CHUNK)], idxs[b], isems[b])

    def fire_gather(b):
        pltpu.async_copy(table_hbm.at[idxs[b]], bufs[b], gsems[b])

    def drain_gather(b):
        pltpu.make_async_copy(table_hbm.at[pl.ds(0, CHUNK)], bufs[b],
                              gsems[b]).wait()

    lane = lax.iota(jnp.int32, LANES)
    sb_vecs = []
    pos_vecs = []
    for u in range(EMBED_DIM // LANES):
        e_vec = lane + u * LANES
        sb_vecs.append(e_vec >> 3)
        pos_vecs.append((e_vec & 7) * CHUNK)

    def transpose(b):
        buf, tr = bufs[b], trs[b]

        def c_step(ci, carry):
            for cu in range(C_UNROLL):
                c = ci * C_UNROLL + cu
                c_splat = jnp.full((LANES,), 0, jnp.int32) + c
                for u in range(EMBED_DIM // LANES):
                    vals = buf[c, pl.ds(u * LANES, LANES)]
                    plsc.store_scatter(
                        tr, [sb_vecs[u], pos_vecs[u] + c_splat], vals)
            return carry

        lax.fori_loop(0, CHUNK // C_UNROLL, c_step, 0)

    def fire_wb(c, b):
        t = c // BLOCKS_PER_T
        vb = c % BLOCKS_PER_T
        pltpu.async_copy(trs[b], out_hbm.at[t, :, vb], wsems[b])

    def drain_wb(b):
        pltpu.make_async_copy(trs[b], out_hbm.at[0, :, 0], wsems[b]).wait()

    for b in range(NBUF):
        fire_idx(c_base + b, b)
    for b in range(NBUF):
        pltpu.make_async_copy(ids_t_hbm.at[0, pl.ds(0, CHUNK)], idxs[b],
                              isems[b]).wait()
        fire_gather(b)

    def group_step(k, carry):
        for b in range(NBUF):
            c = c_base + k * NBUF + b
            drain_gather(b)

            @pl.when(k < N_GROUPS - 1)
            def _():
                fire_idx(c + NBUF, b)

            @pl.when(k > 0)
            def _():
                drain_wb(b)

            transpose(b)
            fire_wb(c, b)

            @pl.when(k < N_GROUPS - 1)
            def _():
                pltpu.make_async_copy(ids_t_hbm.at[0, pl.ds(0, CHUNK)],
                                      idxs[b], isems[b]).wait()
                fire_gather(b)
        return carry

    lax.fori_loop(0, N_GROUPS, group_step, 0)
    for b in range(NBUF):
        drain_wb(b)


def kernel(input_ids, table):
    ids_t = input_ids.T.astype(jnp.int32)
    mesh = plsc.VectorSubcoreMesh(core_axis_name="c", subcore_axis_name="s")
    run = functools.partial(
        pl.kernel,
        mesh=mesh,
        out_type=jax.ShapeDtypeStruct(
            (HIST, EMBED_DIM // 8, BLOCKS_PER_T, 8 * CHUNK), jnp.float32),
        scratch_types=[
            pltpu.VMEM((CHUNK,), jnp.int32),
            pltpu.VMEM((CHUNK,), jnp.int32),
            pltpu.VMEM((CHUNK, EMBED_DIM), jnp.float32),
            pltpu.VMEM((CHUNK, EMBED_DIM), jnp.float32),
            pltpu.VMEM((EMBED_DIM // 8, 8 * CHUNK), jnp.float32),
            pltpu.VMEM((EMBED_DIM // 8, 8 * CHUNK), jnp.float32),
            pltpu.SemaphoreType.DMA,
            pltpu.SemaphoreType.DMA,
            pltpu.SemaphoreType.DMA,
            pltpu.SemaphoreType.DMA,
            pltpu.SemaphoreType.DMA,
            pltpu.SemaphoreType.DMA,
        ],
        compiler_params=pltpu.CompilerParams(use_tc_tiling_on_sc=False, needs_layout_passes=False),
    )(_gather_body)
    out4 = run(ids_t, table)
    out5 = out4.reshape(HIST, EMBED_DIM // 8, BLOCKS_PER_T, 8, CHUNK)
    return out5.transpose(2, 4, 0, 1, 3).reshape(BATCH, HIST, EMBED_DIM)


# speedup vs baseline: 1.5251x; 1.3331x over previous
import functools

import jax
import jax.numpy as jnp
from jax import lax
from jax.experimental import pallas as pl
from jax.experimental.pallas import tpu as pltpu
from jax.experimental.pallas import tpu_sc as plsc

VOCAB = 1000000
EMBED_DIM = 64
BATCH = 4096
HIST = 200

NC, NS = 2, 16
NW = NC * NS
IDX_PER_STREAM = 128
CHUNK = 512
STREAMS_PER_CHUNK = CHUNK // IDX_PER_STREAM
BLOCKS_PER_T = BATCH // CHUNK
N_CHUNKS = HIST * BLOCKS_PER_T
CH_PER_W = N_CHUNKS // NW
NBUF = 2
N_GROUPS = CH_PER_W // NBUF


def _gather_body(ids_t_hbm, table_hbm, out_hbm, idx0, idx1, rows0, rows1,
                 isem0, isem1, gsem0, gsem1, wsem0, wsem1):
    wid = lax.axis_index("s") * NC + lax.axis_index("c")
    c_base = wid * CH_PER_W
    idxs = (idx0, idx1)
    bufs = (rows0, rows1)
    isems = (isem0, isem1)
    gsems = (gsem0, gsem1)
    wsems = (wsem0, wsem1)

    def fire_idx(c, b):
        t = c // BLOCKS_PER_T
        b0 = (c % BLOCKS_PER_T) * CHUNK
        pltpu.async_copy(ids_t_hbm.at[t, pl.ds(b0, CHUNK)], idxs[b], isems[b])

    def fire_gathers(b):
        for j in range(STREAMS_PER_CHUNK):
            pltpu.async_copy(
                table_hbm.at[idxs[b].at[pl.ds(j * IDX_PER_STREAM,
                                              IDX_PER_STREAM)]],
                bufs[b].at[pl.ds(j * IDX_PER_STREAM, IDX_PER_STREAM)],
                gsems[b])

    def drain_gathers(b):
        for j in range(STREAMS_PER_CHUNK):
            pltpu.make_async_copy(
                table_hbm.at[pl.ds(0, IDX_PER_STREAM)],
                bufs[b].at[pl.ds(j * IDX_PER_STREAM, IDX_PER_STREAM)],
                gsems[b]).wait()

    def fire_wb(c, b):
        t = c // BLOCKS_PER_T
        b0 = (c % BLOCKS_PER_T) * CHUNK
        pltpu.async_copy(
            bufs[b],
            out_hbm.at[pl.ds(b0, CHUNK), t // 8, t % 8, pl.ds(0, EMBED_DIM)],
            wsems[b])

    def drain_wb(b):
        pltpu.make_async_copy(
            bufs[b], out_hbm.at[pl.ds(0, CHUNK), 0, 0, pl.ds(0, EMBED_DIM)],
            wsems[b]).wait()

    for b in range(NBUF):
        fire_idx(c_base + b, b)
    for b in range(NBUF):
        pltpu.make_async_copy(ids_t_hbm.at[0, pl.ds(0, CHUNK)], idxs[b],
                              isems[b]).wait()
        fire_gathers(b)

    def group_step(k, carry):
        for b in range(NBUF):
            c = c_base + k * NBUF + b
            drain_gathers(b)

            @pl.when(k < N_GROUPS - 1)
            def _():
                fire_idx(c + NBUF, b)

            fire_wb(c, b)
        for b in range(NBUF):
            drain_wb(b)

            @pl.when(k < N_GROUPS - 1)
            def _():
                pltpu.make_async_copy(ids_t_hbm.at[0, pl.ds(0, CHUNK)],
                                      idxs[b], isems[b]).wait()
                fire_gathers(b)
        return carry

    lax.fori_loop(0, N_GROUPS, group_step, 0)


def kernel(input_ids, table):
    ids_t = input_ids.T.astype(jnp.int32)
    mesh = plsc.VectorSubcoreMesh(core_axis_name="c", subcore_axis_name="s")
    run = functools.partial(
        pl.kernel,
        mesh=mesh,
        out_type=jax.ShapeDtypeStruct((BATCH, HIST // 8, 8, 128),
                                      jnp.float32),
        scratch_types=[
            pltpu.VMEM((CHUNK,), jnp.int32),
            pltpu.VMEM((CHUNK,), jnp.int32),
            pltpu.VMEM((CHUNK, EMBED_DIM), jnp.float32),
            pltpu.VMEM((CHUNK, EMBED_DIM), jnp.float32),
            pltpu.SemaphoreType.DMA,
            pltpu.SemaphoreType.DMA,
            pltpu.SemaphoreType.DMA,
            pltpu.SemaphoreType.DMA,
            pltpu.SemaphoreType.DMA,
            pltpu.SemaphoreType.DMA,
        ],
        compiler_params=pltpu.CompilerParams(use_tc_tiling_on_sc=False,
                                             needs_layout_passes=False),
    )(_gather_body)
    out_raw = run(ids_t, table)
    return out_raw[:, :, :, :EMBED_DIM].reshape(BATCH, HIST, EMBED_DIM)
